# baseline (speedup 1.0000x reference)
import functools

import jax
import jax.numpy as jnp
from jax.experimental import pallas as pl
from jax.experimental.pallas import tpu as pltpu
from jax.experimental.pallas import tpu_sc as plsc

D = 768
E = 8
K = 2
DFF = 3072
T = 2048
TK = T * K

BM = 1024
BF = 1536
NJ = DFF // BF
NT = 12
RMAX = NT * BM

NW = 32
GW = RMAX // NW
GB = 128


def _gmm_body(expert_ref, clampi_ref, xs_ref, wg_ref, wu_ref, wd_ref,
              out_ref, yacc_ref):
    i = pl.program_id(0)
    j = pl.program_id(1)

    @pl.when(clampi_ref[i] == i)
    def _():
        x = xs_ref[...]
        g = jax.lax.dot_general(x, wg_ref[0], (((1,), (1,)), ((), ())),
                                preferred_element_type=jnp.float32)
        u = jax.lax.dot_general(x, wu_ref[0], (((1,), (1,)), ((), ())),
                                preferred_element_type=jnp.float32)
        h = jnp.maximum(g, 0.0) * u
        y = jax.lax.dot_general(h, wd_ref[0], (((1,), (1,)), ((), ())),
                                preferred_element_type=jnp.float32)

        @pl.when(j == 0)
        def _():
            yacc_ref[...] = y

        @pl.when(j > 0)
        def _():
            yacc_ref[...] += y

        @pl.when(j == NJ - 1)
        def _():
            out_ref[...] = yacc_ref[...]


def _grouped_glu(texp, clampi, xs, Wg, Wu, Wd):
    grid_spec = pltpu.PrefetchScalarGridSpec(
        num_scalar_prefetch=2,
        grid=(NT, NJ),
        in_specs=[
            pl.BlockSpec((BM, D), lambda i, j, er, ci: (ci[i], 0)),
            pl.BlockSpec((1, BF, D), lambda i, j, er, ci: (er[i], j, 0)),
            pl.BlockSpec((1, BF, D), lambda i, j, er, ci: (er[i], j, 0)),
            pl.BlockSpec((1, D, BF), lambda i, j, er, ci: (er[i], 0, j)),
        ],
        out_specs=pl.BlockSpec((BM, D), lambda i, j, er, ci: (ci[i], 0)),
        scratch_shapes=[pltpu.VMEM((BM, D), jnp.float32)],
    )
    return pl.pallas_call(
        _gmm_body,
        grid_spec=grid_spec,
        out_shape=jax.ShapeDtypeStruct((RMAX, D), jnp.float32),
        compiler_params=pltpu.CompilerParams(
            dimension_semantics=("arbitrary", "arbitrary"),
        ),
    )(texp, clampi, xs, Wg, Wu, Wd)


def _sc_gather_body(idx_hbm, xf_hbm, xs_hbm, idx_v, rows_v, sem):
    wid = jax.lax.axis_index("s") * 2 + jax.lax.axis_index("c")
    base = wid * GW
    for c in range(GW // GB):
        pltpu.sync_copy(idx_hbm.at[pl.ds(base + c * GB, GB)], idx_v)
        pltpu.async_copy(xf_hbm.at[idx_v], rows_v, sem).wait()
        pltpu.sync_copy(rows_v, xs_hbm.at[pl.ds(base + c * GB, GB)])


def _sc_gather(row_src, xf):
    run = pl.kernel(
        _sc_gather_body,
        mesh=plsc.VectorSubcoreMesh(core_axis_name="c", subcore_axis_name="s"),
        out_type=[jax.ShapeDtypeStruct((RMAX, D), jnp.float32)],
        scratch_types=[
            pltpu.VMEM((GB,), jnp.int32),
            pltpu.VMEM((GB, D), jnp.float32),
            pltpu.SemaphoreType.DMA,
        ],
    )
    out = run(row_src, xf)
    return out[0] if isinstance(out, (list, tuple)) else out


def kernel(x, Wr, Wg, Wu, Wd):
    Bb, Ll, Dd = x.shape
    Tt = Bb * Ll
    TKt = Tt * K
    xf = x.reshape(Tt, Dd)

    logits = xf @ Wr.T
    probs = jax.nn.softmax(logits, axis=-1)
    e1 = jnp.argmax(probs, axis=-1).astype(jnp.int32)
    p1 = jnp.max(probs, axis=-1)
    lane = jnp.arange(E, dtype=jnp.int32)
    probs2 = jnp.where(lane[None, :] == e1[:, None], -jnp.inf, probs)
    e2 = jnp.argmax(probs2, axis=-1).astype(jnp.int32)
    p2 = jnp.max(probs2, axis=-1)
    s = p1 + p2
    k_w = jnp.stack([p1 / s, p2 / s], axis=1)
    e_flat = jnp.concatenate([e1, e2])

    onehot = (e_flat[:, None] == lane[None, :]).astype(jnp.int32)
    csum = jnp.cumsum(onehot, axis=0)
    counts = csum[-1]
    padded = ((counts + BM - 1) // BM) * BM
    gend = jnp.cumsum(padded).astype(jnp.int32)
    gstart = gend - padded
    rank = jnp.take_along_axis(csum, e_flat[:, None], axis=1)[:, 0] - 1
    dest = gstart[e_flat] + rank
    tok = jnp.arange(TKt, dtype=jnp.int32) % Tt
    row_src = jnp.zeros(RMAX, jnp.int32).at[dest].set(tok)

    n_active = gend[-1] // BM
    tiles = jnp.arange(NT, dtype=jnp.int32)
    texp = jnp.sum((tiles[:, None] * BM >= gend[None, :]).astype(jnp.int32),
                   axis=1)
    texp = jnp.minimum(texp, E - 1).astype(jnp.int32)
    clampi = jnp.minimum(tiles, n_active - 1).astype(jnp.int32)
    texp = texp[clampi]

    xs = _sc_gather(row_src, xf)

    y = _grouped_glu(texp, clampi, xs, Wg, Wu, Wd)

    yk = y[dest.reshape(K, Tt)]
    out = jnp.sum(yk * k_w.T[:, :, None], axis=0)
    return out.reshape(Bb, Ll, Dd)


# speedup vs baseline: 1.9895x; 1.9895x over previous
import functools

import jax
import jax.numpy as jnp
from jax.experimental import pallas as pl
from jax.experimental.pallas import tpu as pltpu

D = 768
E = 8
K = 2
DFF = 3072
T = 2048
TK = T * K

BM = 1024
BF = 1536
NJ = DFF // BF
NT = 12
RMAX = NT * BM


def _gmm_body(expert_ref, clampi_ref, xs_ref, wg_ref, wu_ref, wd_ref,
              out_ref, yacc_ref):
    i = pl.program_id(0)
    j = pl.program_id(1)

    @pl.when(clampi_ref[i] == i)
    def _():
        x = xs_ref[...]
        g = jax.lax.dot_general(x, wg_ref[0], (((1,), (1,)), ((), ())),
                                preferred_element_type=jnp.float32)
        u = jax.lax.dot_general(x, wu_ref[0], (((1,), (1,)), ((), ())),
                                preferred_element_type=jnp.float32)
        h = jnp.maximum(g, 0.0) * u
        y = jax.lax.dot_general(h, wd_ref[0], (((1,), (1,)), ((), ())),
                                preferred_element_type=jnp.float32)

        @pl.when(j == 0)
        def _():
            yacc_ref[...] = y

        @pl.when(j > 0)
        def _():
            yacc_ref[...] += y

        @pl.when(j == NJ - 1)
        def _():
            out_ref[...] = yacc_ref[...]


def _grouped_glu(texp, clampi, xs, Wg, Wu, Wd):
    grid_spec = pltpu.PrefetchScalarGridSpec(
        num_scalar_prefetch=2,
        grid=(NT, NJ),
        in_specs=[
            pl.BlockSpec((BM, D), lambda i, j, er, ci: (ci[i], 0)),
            pl.BlockSpec((1, BF, D), lambda i, j, er, ci: (er[i], j, 0)),
            pl.BlockSpec((1, BF, D), lambda i, j, er, ci: (er[i], j, 0)),
            pl.BlockSpec((1, D, BF), lambda i, j, er, ci: (er[i], 0, j)),
        ],
        out_specs=pl.BlockSpec((BM, D), lambda i, j, er, ci: (ci[i], 0)),
        scratch_shapes=[pltpu.VMEM((BM, D), jnp.float32)],
    )
    return pl.pallas_call(
        _gmm_body,
        grid_spec=grid_spec,
        out_shape=jax.ShapeDtypeStruct((RMAX, D), jnp.float32),
        compiler_params=pltpu.CompilerParams(
            dimension_semantics=("arbitrary", "arbitrary"),
        ),
    )(texp, clampi, xs, Wg, Wu, Wd)


def kernel(x, Wr, Wg, Wu, Wd):
    Bb, Ll, Dd = x.shape
    Tt = Bb * Ll
    TKt = Tt * K
    xf = x.reshape(Tt, Dd)

    logits = xf @ Wr.T
    probs = jax.nn.softmax(logits, axis=-1)
    e1 = jnp.argmax(probs, axis=-1).astype(jnp.int32)
    p1 = jnp.max(probs, axis=-1)
    lane = jnp.arange(E, dtype=jnp.int32)
    probs2 = jnp.where(lane[None, :] == e1[:, None], -jnp.inf, probs)
    e2 = jnp.argmax(probs2, axis=-1).astype(jnp.int32)
    p2 = jnp.max(probs2, axis=-1)
    s = p1 + p2
    k_w = jnp.stack([p1 / s, p2 / s], axis=1)
    e_flat = jnp.concatenate([e1, e2])

    onehot = (e_flat[:, None] == lane[None, :]).astype(jnp.int32)
    csum = jnp.cumsum(onehot, axis=0)
    counts = csum[-1]
    padded = ((counts + BM - 1) // BM) * BM
    gend = jnp.cumsum(padded).astype(jnp.int32)
    gstart = gend - padded
    rank = jnp.take_along_axis(csum, e_flat[:, None], axis=1)[:, 0] - 1
    dest = gstart[e_flat] + rank
    tok = jnp.arange(TKt, dtype=jnp.int32) % Tt
    row_src = jnp.zeros(RMAX, jnp.int32).at[dest].set(tok)

    n_active = gend[-1] // BM
    tiles = jnp.arange(NT, dtype=jnp.int32)
    texp = jnp.sum((tiles[:, None] * BM >= gend[None, :]).astype(jnp.int32),
                   axis=1)
    texp = jnp.minimum(texp, E - 1).astype(jnp.int32)
    clampi = jnp.minimum(tiles, n_active - 1).astype(jnp.int32)
    texp = texp[clampi]

    xs = xf[row_src]

    y = _grouped_glu(texp, clampi, xs, Wg, Wu, Wd)

    yk = y[dest.reshape(K, Tt)]
    out = jnp.sum(yk * k_w.T[:, :, None], axis=0)
    return out.reshape(Bb, Ll, Dd)
